# baseline (speedup 1.0000x reference)
import math

import jax
import jax.numpy as jnp
from jax.experimental import pallas as pl
from jax.experimental.pallas import tpu as pltpu

_C_GELU = math.sqrt(2.0 / math.pi)
_R = 1024
_N = 6


def _gelu(v):
    inner = _C_GELU * (v + 0.044715 * (v * v * v))
    return 0.5 * v * (1.0 + jnp.tanh(inner))


def _gelu_pipe(x_hbm, y_hbm, in_buf, out_buf, load_sem, store_sem):
    n_chunks = x_hbm.shape[0] // _R

    def _start_load(i, slot):
        pltpu.make_async_copy(
            x_hbm.at[pl.ds(i * _R, _R), :], in_buf.at[slot], load_sem.at[slot]
        ).start()

    for s in range(_N):
        _start_load(s, s)

    def body(i, carry):
        slot = jax.lax.rem(i, _N)
        pltpu.make_async_copy(
            x_hbm.at[pl.ds(i * _R, _R), :], in_buf.at[slot], load_sem.at[slot]
        ).wait()

        @pl.when(i >= _N)
        def _():
            pltpu.make_async_copy(
                out_buf.at[slot],
                y_hbm.at[pl.ds((i - _N) * _R, _R), :],
                store_sem.at[slot],
            ).wait()

        out_buf[slot] = _gelu(in_buf[slot])
        pltpu.make_async_copy(
            out_buf.at[slot], y_hbm.at[pl.ds(i * _R, _R), :], store_sem.at[slot]
        ).start()

        @pl.when(i + _N < n_chunks)
        def _():
            _start_load(i + _N, slot)

        return carry

    jax.lax.fori_loop(0, n_chunks, body, 0, unroll=False)

    for s in range(_N):
        i_last = n_chunks - _N + s
        pltpu.make_async_copy(
            out_buf.at[s], y_hbm.at[pl.ds(i_last * _R, _R), :], store_sem.at[s]
        ).wait()


def kernel(x):
    B, T, D = x.shape
    rows = B * T
    x2 = x.reshape(rows, D)
    y2 = pl.pallas_call(
        _gelu_pipe,
        in_specs=[pl.BlockSpec(memory_space=pltpu.MemorySpace.HBM)],
        out_specs=pl.BlockSpec(memory_space=pltpu.MemorySpace.HBM),
        out_shape=jax.ShapeDtypeStruct((rows, D), x.dtype),
        scratch_shapes=[
            pltpu.VMEM((_N, _R, D), x.dtype),
            pltpu.VMEM((_N, _R, D), x.dtype),
            pltpu.SemaphoreType.DMA((_N,)),
            pltpu.SemaphoreType.DMA((_N,)),
        ],
        compiler_params=pltpu.CompilerParams(
            vmem_limit_bytes=100 * 1024 * 1024,
        ),
    )(x2)
    return y2.reshape(B, T, D)


# speedup vs baseline: 1.0038x; 1.0038x over previous
import math

import jax
import jax.numpy as jnp
from jax.experimental import pallas as pl
from jax.experimental.pallas import tpu as pltpu

_C_GELU = math.sqrt(2.0 / math.pi)
_R = 2048
_N = 3


def _gelu(v):
    inner = _C_GELU * (v + 0.044715 * (v * v * v))
    return 0.5 * v * (1.0 + jnp.tanh(inner))


def _gelu_pipe(x_hbm, y_hbm, in_buf, out_buf, load_sem, store_sem):
    n_chunks = x_hbm.shape[0] // _R

    _H = _R // 2

    def _load_half(i, slot, h):
        return pltpu.make_async_copy(
            x_hbm.at[pl.ds(i * _R + h * _H, _H), :],
            in_buf.at[slot, pl.ds(h * _H, _H), :],
            load_sem.at[slot, h],
        )

    def _store_half(i, slot, h):
        return pltpu.make_async_copy(
            out_buf.at[slot, pl.ds(h * _H, _H), :],
            y_hbm.at[pl.ds(i * _R + h * _H, _H), :],
            store_sem.at[slot, h],
        )

    def _start_load(i, slot):
        _load_half(i, slot, 0).start()
        _load_half(i, slot, 1).start()

    for s in range(_N):
        _start_load(s, s)

    def body(i, carry):
        slot = jax.lax.rem(i, _N)
        _load_half(i, slot, 0).wait()
        _load_half(i, slot, 1).wait()

        @pl.when(i >= _N)
        def _():
            _store_half(i - _N, slot, 0).wait()
            _store_half(i - _N, slot, 1).wait()

        out_buf[slot] = _gelu(in_buf[slot])
        _store_half(i, slot, 0).start()
        _store_half(i, slot, 1).start()

        @pl.when(i + _N < n_chunks)
        def _():
            _start_load(i + _N, slot)

        return carry

    jax.lax.fori_loop(0, n_chunks, body, 0, unroll=False)

    for s in range(_N):
        i_last = n_chunks - _N + s
        _store_half(i_last, s, 0).wait()
        _store_half(i_last, s, 1).wait()


def kernel(x):
    B, T, D = x.shape
    rows = B * T
    x2 = x.reshape(rows, D)
    y2 = pl.pallas_call(
        _gelu_pipe,
        in_specs=[pl.BlockSpec(memory_space=pltpu.MemorySpace.HBM)],
        out_specs=pl.BlockSpec(memory_space=pltpu.MemorySpace.HBM),
        out_shape=jax.ShapeDtypeStruct((rows, D), x.dtype),
        scratch_shapes=[
            pltpu.VMEM((_N, _R, D), x.dtype),
            pltpu.VMEM((_N, _R, D), x.dtype),
            pltpu.SemaphoreType.DMA((_N, 2)),
            pltpu.SemaphoreType.DMA((_N, 2)),
        ],
        compiler_params=pltpu.CompilerParams(
            vmem_limit_bytes=100 * 1024 * 1024,
        ),
    )(x2)
    return y2.reshape(B, T, D)


# speedup vs baseline: 1.0039x; 1.0001x over previous
import math

import jax
import jax.numpy as jnp
from jax.experimental import pallas as pl
from jax.experimental.pallas import tpu as pltpu

_C_GELU = math.sqrt(2.0 / math.pi)
_R = 2048
_N = 3


def _gelu(v):
    inner = _C_GELU * (v + 0.044715 * (v * v * v))
    return 0.5 * v * (1.0 + jnp.tanh(inner))


def _gelu_pipe(x_hbm, y_hbm, in_buf, out_buf, load_sem, store_sem):
    n_chunks = x_hbm.shape[0] // _R

    def _load(i, slot):
        return pltpu.make_async_copy(
            x_hbm.at[pl.ds(i * _R, _R), :], in_buf.at[slot], load_sem.at[slot]
        )

    def _store(i, slot):
        return pltpu.make_async_copy(
            out_buf.at[slot], y_hbm.at[pl.ds(i * _R, _R), :], store_sem.at[slot]
        )

    for s in range(_N):
        _load(s, s).start()

    def body(i, carry):
        slot = jax.lax.rem(i, _N)
        _load(i, slot).wait()

        @pl.when(i >= _N)
        def _():
            _store(i - _N, slot).wait()

        out_buf[slot] = _gelu(in_buf[slot])
        _store(i, slot).start()

        @pl.when(i + _N < n_chunks)
        def _():
            _load(i + _N, slot).start()

        return carry

    jax.lax.fori_loop(0, n_chunks, body, 0, unroll=False)

    for s in range(_N):
        i_last = n_chunks - _N + s
        _store(i_last, s).wait()


def kernel(x):
    B, T, D = x.shape
    rows = B * T
    x2 = x.reshape(rows, D)
    y2 = pl.pallas_call(
        _gelu_pipe,
        in_specs=[pl.BlockSpec(memory_space=pltpu.MemorySpace.HBM)],
        out_specs=pl.BlockSpec(memory_space=pltpu.MemorySpace.HBM),
        out_shape=jax.ShapeDtypeStruct((rows, D), x.dtype),
        scratch_shapes=[
            pltpu.VMEM((_N, _R, D), x.dtype),
            pltpu.VMEM((_N, _R, D), x.dtype),
            pltpu.SemaphoreType.DMA((_N,)),
            pltpu.SemaphoreType.DMA((_N,)),
        ],
        compiler_params=pltpu.CompilerParams(
            vmem_limit_bytes=100 * 1024 * 1024,
        ),
    )(x2)
    return y2.reshape(B, T, D)


# speedup vs baseline: 1.0061x; 1.0022x over previous
import math

import jax
import jax.numpy as jnp
from jax.experimental import pallas as pl
from jax.experimental.pallas import tpu as pltpu

_C_GELU = math.sqrt(2.0 / math.pi)
_R = 2048
_N = 3


def _gelu(v):
    inner = _C_GELU * (v + 0.044715 * (v * v * v))
    return 0.5 * v * (1.0 + jnp.tanh(inner))


def _gelu_pipe(x_hbm, y_hbm, in_buf, out_buf, load_sem, store_sem):
    n_chunks = x_hbm.shape[0] // _R

    def _load(i, slot):
        return pltpu.make_async_copy(
            x_hbm.at[pl.ds(i * _R, _R), :], in_buf.at[slot], load_sem.at[slot]
        )

    def _store(i, slot):
        return pltpu.make_async_copy(
            out_buf.at[slot], y_hbm.at[pl.ds(i * _R, _R), :], store_sem.at[slot]
        )

    for s in range(_N):
        _load(s, s).start()

    def body(i, carry):
        slot = jax.lax.rem(i, _N)
        _load(i, slot).wait()

        @pl.when(i >= _N)
        def _():
            _store(i - _N, slot).wait()

        out_buf[slot] = _gelu(in_buf[slot])
        _store(i, slot).start()

        @pl.when(i + _N < n_chunks)
        def _():
            _load(i + _N, slot).start()

        return carry

    jax.lax.fori_loop(0, n_chunks, body, 0, unroll=False)

    for i_last in range(n_chunks - _N, n_chunks):
        _store(i_last, i_last % _N).wait()


def kernel(x):
    B, T, D = x.shape
    rows = B * T
    x2 = x.reshape(rows, D)
    y2 = pl.pallas_call(
        _gelu_pipe,
        in_specs=[pl.BlockSpec(memory_space=pltpu.MemorySpace.HBM)],
        out_specs=pl.BlockSpec(memory_space=pltpu.MemorySpace.HBM),
        out_shape=jax.ShapeDtypeStruct((rows, D), x.dtype),
        scratch_shapes=[
            pltpu.VMEM((_N, _R, D), x.dtype),
            pltpu.VMEM((_N, _R, D), x.dtype),
            pltpu.SemaphoreType.DMA((_N,)),
            pltpu.SemaphoreType.DMA((_N,)),
        ],
        compiler_params=pltpu.CompilerParams(
            vmem_limit_bytes=100 * 1024 * 1024,
        ),
    )(x2)
    return y2.reshape(B, T, D)
